# baseline (speedup 1.0000x reference)
import functools

import jax
import jax.numpy as jnp
from jax import lax
from jax.experimental import pallas as pl
from jax.experimental.pallas import tpu as pltpu
from jax.experimental.pallas import tpu_sc as plsc

N = 16384

R_SC = 3584
R_TC = N - R_SC

BR = 256
NR = R_TC // BR


def _tc_body(m0_ref, m1_ref, m2_ref, m3_ref, v_ref, r_ref, out_ref, loss_ref, max_ref):
    i = pl.program_id(0)
    v = v_ref[...]
    q = N // 4
    row = jnp.sum(m0_ref[...] * v[:, 0 * q : 1 * q], axis=1)
    row = row + jnp.sum(m1_ref[...] * v[:, 1 * q : 2 * q], axis=1)
    row = row + jnp.sum(m2_ref[...] * v[:, 2 * q : 3 * q], axis=1)
    row = row + jnp.sum(m3_ref[...] * v[:, 3 * q : 4 * q], axis=1)
    out_ref[0, :] = row
    err = row - r_ref[0, :]
    s = jnp.sum(err * err, keepdims=True).reshape(1, 1)
    a = jnp.max(jnp.abs(err), keepdims=True).reshape(1, 1)

    @pl.when(i == 0)
    def _init():
        loss_ref[...] = s
        max_ref[...] = a

    @pl.when(i > 0)
    def _acc():
        loss_ref[...] = loss_ref[...] + s
        max_ref[...] = jnp.maximum(max_ref[...], a)


def _tc_call(matrix, v2, r2):
    return pl.pallas_call(
        _tc_body,
        grid=(NR,),
        in_specs=[
            pl.BlockSpec((BR, N // 4), lambda i: (i, 0)),
            pl.BlockSpec((BR, N // 4), lambda i: (i, 1)),
            pl.BlockSpec((BR, N // 4), lambda i: (i, 2)),
            pl.BlockSpec((BR, N // 4), lambda i: (i, 3)),
            pl.BlockSpec((1, N), lambda i: (0, 0)),
            pl.BlockSpec((1, BR), lambda i: (0, i)),
        ],
        out_specs=[
            pl.BlockSpec((1, BR), lambda i: (0, i)),
            pl.BlockSpec((1, 1), lambda i: (0, 0)),
            pl.BlockSpec((1, 1), lambda i: (0, 0)),
        ],
        out_shape=[
            jax.ShapeDtypeStruct((1, R_TC), jnp.float32),
            jax.ShapeDtypeStruct((1, 1), jnp.float32),
            jax.ShapeDtypeStruct((1, 1), jnp.float32),
        ],
    )(matrix, matrix, matrix, matrix, v2, r2)


NC = 2
NS = 16
NW = NC * NS
C = R_SC // NW
G = 2
NG = C // G
CH = N // 16

_sc_mesh = plsc.VectorSubcoreMesh(
    core_axis_name="c", subcore_axis_name="s", num_cores=NC, num_subcores=NS
)


@functools.partial(
    pl.kernel,
    out_type=[
        jax.ShapeDtypeStruct((R_SC,), jnp.float32),
        jax.ShapeDtypeStruct((NW, 16), jnp.float32),
        jax.ShapeDtypeStruct((NW, 16), jnp.float32),
    ],
    mesh=_sc_mesh,
    scratch_types=[
        pltpu.VMEM((N,), jnp.float32),
        pltpu.VMEM((G, N), jnp.float32),
        pltpu.VMEM((G, N), jnp.float32),
        pltpu.VMEM((C,), jnp.float32),
        pltpu.VMEM((C,), jnp.float32),
        pltpu.VMEM((2, 16), jnp.float32),
        pltpu.SemaphoreType.DMA,
        pltpu.SemaphoreType.DMA,
    ],
)
def _sc_call(m_hbm, v_hbm, r_hbm, y_hbm, lp_hbm, mp_hbm,
             v_v, buf0, buf1, refc, outc, stat_v, sem0, sem1):
    wid = lax.axis_index("s") * NC + lax.axis_index("c")
    base = R_TC + wid * C
    pltpu.sync_copy(v_hbm, v_v)
    pltpu.sync_copy(r_hbm.at[pl.ds(base, C)], refc)

    zero = jnp.zeros((16,), jnp.float32)
    lanes = jnp.arange(16, dtype=jnp.int32)

    def dot_group(buf):
        def chunkstep(j, carry):
            a0, a1 = carry
            o = j * 16
            vv = v_v[pl.ds(o, 16)]
            a0 = a0 + buf[0, pl.ds(o, 16)] * vv
            a1 = a1 + buf[1, pl.ds(o, 16)] * vv
            return (a0, a1)

        a0, a1 = lax.fori_loop(0, CH, chunkstep, (zero, zero), unroll=8)
        s0 = a0[0]
        s1 = a1[0]
        for t in range(1, 16):
            s0 = s0 + a0[t]
            s1 = s1 + a1[t]
        return s0, s1

    def issue(buf, sem, g):
        return pltpu.make_async_copy(
            m_hbm.at[pl.ds(base + g * G, G), :], buf, sem
        )

    issue(buf0, sem0, 0).start()
    issue(buf1, sem1, 1).start()

    bufs = (buf0, buf1)
    sems = (sem0, sem1)
    GPS = 16 // G

    def supergroup(s, carry):
        lacc, macc = carry
        g0 = s * GPS
        y = zero
        for k in range(GPS):
            b, sm = bufs[k % 2], sems[k % 2]
            issue(b, sm, g0 + k).wait()
            s0, s1 = dot_group(b)
            y = jnp.where(lanes == 2 * k, s0, y)
            y = jnp.where(lanes == 2 * k + 1, s1, y)

            @pl.when(g0 + k + 2 < NG)
            def _next():
                issue(b, sm, g0 + k + 2).start()

        i0 = s * 16
        outc[pl.ds(i0, 16)] = y
        err = y - refc[pl.ds(i0, 16)]
        return (lacc + err * err, jnp.maximum(macc, jnp.abs(err)))

    lacc, macc = lax.fori_loop(0, C // 16, supergroup, (zero, zero))
    stat_v[0, :] = lacc
    stat_v[1, :] = macc
    pltpu.sync_copy(stat_v.at[0], lp_hbm.at[wid])
    pltpu.sync_copy(stat_v.at[1], mp_hbm.at[wid])
    pltpu.sync_copy(outc, y_hbm.at[pl.ds(wid * C, C)])


@jax.jit
def _run(matrix, vector, ref):
    v2 = vector.reshape(1, N)
    r2 = ref.reshape(1, N)
    y_sc, lp, mp = _sc_call(matrix, vector.reshape(N), ref)
    out_tc, loss_tc, max_tc = _tc_call(matrix, v2, r2)
    out = jnp.concatenate([out_tc.reshape(R_TC), y_sc])
    loss = (loss_tc[0, 0] + jnp.sum(lp)) * (1.0 / N)
    mabs = jnp.maximum(max_tc[0, 0], jnp.max(mp))
    return loss, out, mabs


def kernel(matrix, vector, ref):
    loss, out, mabs = _run(matrix, vector, ref)
    return (loss, out, ref, mabs)


# speedup vs baseline: 1.0210x; 1.0210x over previous
import functools

import jax
import jax.numpy as jnp
from jax import lax
from jax.experimental import pallas as pl
from jax.experimental.pallas import tpu as pltpu
from jax.experimental.pallas import tpu_sc as plsc

N = 16384

R_SC = 1024
R_TC = N - R_SC

BR = 256
NR = R_TC // BR


def _tc_body(m0_ref, m1_ref, m2_ref, m3_ref, v_ref, r_ref, out_ref, loss_ref, max_ref):
    i = pl.program_id(0)
    v = v_ref[...]
    q = N // 4
    row = jnp.sum(m0_ref[...] * v[:, 0 * q : 1 * q], axis=1)
    row = row + jnp.sum(m1_ref[...] * v[:, 1 * q : 2 * q], axis=1)
    row = row + jnp.sum(m2_ref[...] * v[:, 2 * q : 3 * q], axis=1)
    row = row + jnp.sum(m3_ref[...] * v[:, 3 * q : 4 * q], axis=1)
    out_ref[0, :] = row
    err = row - r_ref[0, :]
    s = jnp.sum(err * err, keepdims=True).reshape(1, 1)
    a = jnp.max(jnp.abs(err), keepdims=True).reshape(1, 1)

    @pl.when(i == 0)
    def _init():
        loss_ref[...] = s
        max_ref[...] = a

    @pl.when(i > 0)
    def _acc():
        loss_ref[...] = loss_ref[...] + s
        max_ref[...] = jnp.maximum(max_ref[...], a)


def _tc_call(matrix, v2, r2):
    return pl.pallas_call(
        _tc_body,
        grid=(NR,),
        in_specs=[
            pl.BlockSpec((BR, N // 4), lambda i: (i, 0)),
            pl.BlockSpec((BR, N // 4), lambda i: (i, 1)),
            pl.BlockSpec((BR, N // 4), lambda i: (i, 2)),
            pl.BlockSpec((BR, N // 4), lambda i: (i, 3)),
            pl.BlockSpec((1, N), lambda i: (0, 0)),
            pl.BlockSpec((1, BR), lambda i: (0, i)),
        ],
        out_specs=[
            pl.BlockSpec((1, BR), lambda i: (0, i)),
            pl.BlockSpec((1, 1), lambda i: (0, 0)),
            pl.BlockSpec((1, 1), lambda i: (0, 0)),
        ],
        out_shape=[
            jax.ShapeDtypeStruct((1, R_TC), jnp.float32),
            jax.ShapeDtypeStruct((1, 1), jnp.float32),
            jax.ShapeDtypeStruct((1, 1), jnp.float32),
        ],
    )(matrix, matrix, matrix, matrix, v2, r2)


NC = 2
NS = 16
NW = NC * NS
C = R_SC // NW
G = 2
NG = C // G
CH = N // 16

_sc_mesh = plsc.VectorSubcoreMesh(
    core_axis_name="c", subcore_axis_name="s", num_cores=NC, num_subcores=NS
)


@functools.partial(
    pl.kernel,
    out_type=[
        jax.ShapeDtypeStruct((R_SC,), jnp.float32),
        jax.ShapeDtypeStruct((NW, 16), jnp.float32),
        jax.ShapeDtypeStruct((NW, 16), jnp.float32),
    ],
    mesh=_sc_mesh,
    scratch_types=[
        pltpu.VMEM((N,), jnp.float32),
        pltpu.VMEM((G, N), jnp.float32),
        pltpu.VMEM((G, N), jnp.float32),
        pltpu.VMEM((C,), jnp.float32),
        pltpu.VMEM((C,), jnp.float32),
        pltpu.VMEM((2, 16), jnp.float32),
        pltpu.SemaphoreType.DMA,
        pltpu.SemaphoreType.DMA,
    ],
)
def _sc_call(m_hbm, v_hbm, r_hbm, y_hbm, lp_hbm, mp_hbm,
             v_v, buf0, buf1, refc, outc, stat_v, sem0, sem1):
    wid = lax.axis_index("s") * NC + lax.axis_index("c")
    base = R_TC + wid * C
    pltpu.sync_copy(v_hbm, v_v)
    pltpu.sync_copy(r_hbm.at[pl.ds(base, C)], refc)

    zero = jnp.zeros((16,), jnp.float32)
    lanes = jnp.arange(16, dtype=jnp.int32)

    def dot_group(buf):
        def chunkstep(j, carry):
            a0, a1 = carry
            o = j * 16
            vv = v_v[pl.ds(o, 16)]
            a0 = a0 + buf[0, pl.ds(o, 16)] * vv
            a1 = a1 + buf[1, pl.ds(o, 16)] * vv
            return (a0, a1)

        a0, a1 = lax.fori_loop(0, CH, chunkstep, (zero, zero), unroll=8)
        s0 = a0[0]
        s1 = a1[0]
        for t in range(1, 16):
            s0 = s0 + a0[t]
            s1 = s1 + a1[t]
        return s0, s1

    def issue(buf, sem, g):
        return pltpu.make_async_copy(
            m_hbm.at[pl.ds(base + g * G, G), :], buf, sem
        )

    issue(buf0, sem0, 0).start()
    issue(buf1, sem1, 1).start()

    bufs = (buf0, buf1)
    sems = (sem0, sem1)
    GPS = 16 // G

    def supergroup(s, carry):
        lacc, macc = carry
        g0 = s * GPS
        y = zero
        for k in range(GPS):
            b, sm = bufs[k % 2], sems[k % 2]
            issue(b, sm, g0 + k).wait()
            s0, s1 = dot_group(b)
            y = jnp.where(lanes == 2 * k, s0, y)
            y = jnp.where(lanes == 2 * k + 1, s1, y)

            @pl.when(g0 + k + 2 < NG)
            def _next():
                issue(b, sm, g0 + k + 2).start()

        i0 = s * 16
        outc[pl.ds(i0, 16)] = y
        err = y - refc[pl.ds(i0, 16)]
        return (lacc + err * err, jnp.maximum(macc, jnp.abs(err)))

    lacc, macc = lax.fori_loop(0, C // 16, supergroup, (zero, zero))
    stat_v[0, :] = lacc
    stat_v[1, :] = macc
    pltpu.sync_copy(stat_v.at[0], lp_hbm.at[wid])
    pltpu.sync_copy(stat_v.at[1], mp_hbm.at[wid])
    pltpu.sync_copy(outc, y_hbm.at[pl.ds(wid * C, C)])


@jax.jit
def _run(matrix, vector, ref):
    v2 = vector.reshape(1, N)
    r2 = ref.reshape(1, N)
    y_sc, lp, mp = _sc_call(matrix, vector.reshape(N), ref)
    out_tc, loss_tc, max_tc = _tc_call(matrix, v2, r2)
    out = jnp.concatenate([out_tc.reshape(R_TC), y_sc])
    loss = (loss_tc[0, 0] + jnp.sum(lp)) * (1.0 / N)
    mabs = jnp.maximum(max_tc[0, 0], jnp.max(mp))
    return loss, out, mabs


def kernel(matrix, vector, ref):
    loss, out, mabs = _run(matrix, vector, ref)
    return (loss, out, ref, mabs)


# speedup vs baseline: 1.1122x; 1.0894x over previous
import jax
import jax.numpy as jnp
from jax.experimental import pallas as pl

N = 16384
BR = 256
NR = N // BR


def _body(m0_ref, m1_ref, m2_ref, m3_ref, v_ref, r_ref, out_ref, loss_ref, max_ref):
    i = pl.program_id(0)
    v = v_ref[...]
    q = N // 4
    row = jnp.sum(m0_ref[...] * v[:, 0 * q : 1 * q], axis=1)
    row = row + jnp.sum(m1_ref[...] * v[:, 1 * q : 2 * q], axis=1)
    row = row + jnp.sum(m2_ref[...] * v[:, 2 * q : 3 * q], axis=1)
    row = row + jnp.sum(m3_ref[...] * v[:, 3 * q : 4 * q], axis=1)
    out_ref[0, :] = row
    err = row - r_ref[0, :]
    s = jnp.sum(err * err, keepdims=True).reshape(1, 1)
    a = jnp.max(jnp.abs(err), keepdims=True).reshape(1, 1)

    @pl.when(i == 0)
    def _init():
        loss_ref[...] = s
        max_ref[...] = a

    @pl.when(i > 0)
    def _acc():
        loss_ref[...] = loss_ref[...] + s
        max_ref[...] = jnp.maximum(max_ref[...], a)

    @pl.when(i == NR - 1)
    def _fin():
        loss_ref[...] = loss_ref[...] * (1.0 / N)


@jax.jit
def _run(matrix, vector, ref):
    v2 = vector.reshape(1, N)
    r2 = ref.reshape(1, N)
    out, loss, mabs = pl.pallas_call(
        _body,
        grid=(NR,),
        in_specs=[
            pl.BlockSpec((BR, N // 4), lambda i: (i, 0)),
            pl.BlockSpec((BR, N // 4), lambda i: (i, 1)),
            pl.BlockSpec((BR, N // 4), lambda i: (i, 2)),
            pl.BlockSpec((BR, N // 4), lambda i: (i, 3)),
            pl.BlockSpec((1, N), lambda i: (0, 0)),
            pl.BlockSpec((1, BR), lambda i: (0, i)),
        ],
        out_specs=[
            pl.BlockSpec((1, BR), lambda i: (0, i)),
            pl.BlockSpec((1, 1), lambda i: (0, 0)),
            pl.BlockSpec((1, 1), lambda i: (0, 0)),
        ],
        out_shape=[
            jax.ShapeDtypeStruct((1, N), jnp.float32),
            jax.ShapeDtypeStruct((1, 1), jnp.float32),
            jax.ShapeDtypeStruct((1, 1), jnp.float32),
        ],
    )(matrix, matrix, matrix, matrix, v2, r2)
    return loss[0, 0], out.reshape(N), mabs[0, 0]


def kernel(matrix, vector, ref):
    loss, out, mabs = _run(matrix, vector, ref)
    return (loss, out, ref, mabs)
